# baseline (speedup 1.0000x reference)
import functools

import jax
import jax.numpy as jnp
from jax import lax
from jax.experimental import pallas as pl
from jax.experimental.pallas import tpu as pltpu
from jax.experimental.pallas import tpu_sc as plsc

_NC, _NS = 2, 16
_NW = _NC * _NS
_KCH = 400
_BK = 16000


def _sc_gather(xp, pd, src, dst):
    e = src.shape[0]
    e_per_w = e // _NW
    nch = e_per_w // _KCH
    mesh = plsc.VectorSubcoreMesh(core_axis_name="c", subcore_axis_name="s")

    @functools.partial(
        pl.kernel,
        out_type=jax.ShapeDtypeStruct((e, 64), jnp.float32),
        mesh=mesh,
        compiler_params=pltpu.CompilerParams(use_tc_tiling_on_sc=False),
        scratch_types=[
            pltpu.VMEM((_KCH,), jnp.int32),
            pltpu.VMEM((_KCH,), jnp.int32),
            pltpu.VMEM((_KCH, 48), jnp.float32),
            pltpu.VMEM((_KCH, 16), jnp.float32),
            pltpu.SemaphoreType.DMA,
            pltpu.SemaphoreType.DMA,
        ],
    )
    def k(xp_hbm, pd_hbm, src_hbm, dst_hbm, g_out,
          sidx, didx, srows, drows, sem1, sem2):
        wid = lax.axis_index("s") * _NC + lax.axis_index("c")
        base = wid * e_per_w

        def body(j, carry):
            off = base + j * _KCH
            pltpu.sync_copy(src_hbm.at[pl.ds(off, _KCH)], sidx)
            pltpu.sync_copy(dst_hbm.at[pl.ds(off, _KCH)], didx)
            cp1 = pltpu.async_copy(xp_hbm.at[sidx], srows, sem1)
            cp2 = pltpu.async_copy(pd_hbm.at[didx], drows, sem2)
            cp1.wait()
            cp2.wait()
            pltpu.sync_copy(srows, g_out.at[pl.ds(off, _KCH), pl.ds(0, 48)])
            pltpu.sync_copy(drows, g_out.at[pl.ds(off, _KCH), pl.ds(48, 16)])
            return carry

        lax.fori_loop(0, nch, body, 0)

    return k(xp, pd, src, dst)


def _edge_mlp_body(g_ref, w1_ref, b1_ref, w2_ref, b2_ref, o_ref):
    g = jnp.maximum(
        jnp.dot(g_ref[...], w1_ref[...], preferred_element_type=jnp.float32)
        + b1_ref[...], 0.0)
    o = jnp.dot(g, w2_ref[...], preferred_element_type=jnp.float32)
    o_ref[...] = jnp.maximum(o + b2_ref[...], 0.0).astype(jnp.bfloat16)


def _edge_mlp(g, Wcat, ba, Wb, bb):
    e = g.shape[0]
    return pl.pallas_call(
        _edge_mlp_body,
        grid=(e // _BK,),
        in_specs=[
            pl.BlockSpec((_BK, 64), lambda i: (i, 0)),
            pl.BlockSpec((64, 64), lambda i: (0, 0)),
            pl.BlockSpec((1, 64), lambda i: (0, 0)),
            pl.BlockSpec((64, 128), lambda i: (0, 0)),
            pl.BlockSpec((1, 128), lambda i: (0, 0)),
        ],
        out_specs=pl.BlockSpec((_BK, 128), lambda i: (i, 0)),
        out_shape=jax.ShapeDtypeStruct((e, 128), jnp.bfloat16),
    )(g, Wcat, ba.reshape(1, 64), Wb, bb.reshape(1, 128))


def _point_conv(xp, pd, edge_index, Wa, ba, Wb, bb, n):
    src = edge_index[0]
    dst = edge_index[1]
    e = src.shape[0]
    step = _NW * _KCH
    e_pad = ((e + step - 1) // step) * step
    dst_seg = dst
    if e_pad != e:
        src = jnp.pad(src, (0, e_pad - e))
        dst = jnp.pad(dst, (0, e_pad - e))
        dst_seg = jnp.pad(dst_seg, (0, e_pad - e), constant_values=n)
    Wcat = jnp.zeros((64, 64), jnp.float32)
    Wcat = Wcat.at[0:32].set(Wa[:32])
    Wcat = Wcat.at[32:35].set(Wa[32:])
    Wcat = Wcat.at[48:51].set(-Wa[32:])
    g = _sc_gather(xp, pd, src, dst)
    msg = _edge_mlp(g, Wcat, ba, Wb, bb)
    out = jax.ops.segment_max(msg, dst_seg, num_segments=n + 1)[:n]
    out = out.astype(jnp.float32)
    return jnp.where(jnp.isfinite(out), out, 0.0)


def kernel(x, pos, edge_index1, edge_index2, residue_index, src_idx, tgt_idx,
           y_raw, W1a, b1a, W1b, b1b, W2a, b2a, W2b, b2b, lin1_W, lin1_b,
           lin2_W, lin2_b):
    n = x.shape[0]
    xp = jnp.concatenate([x, pos, jnp.zeros((n, 13), jnp.float32)], axis=1)
    pd = jnp.concatenate([pos, jnp.zeros((n, 13), jnp.float32)], axis=1)
    x1 = _point_conv(xp, pd, edge_index1, W1a, b1a, W1b, b1b, n)
    x2 = _point_conv(xp, pd, edge_index2, W2a, b2a, W2b, b2b, n)
    h = jnp.concatenate([x1, x2, jnp.ones((n, 1), jnp.float32)], axis=1)
    r = 10000
    s_all = jax.ops.segment_sum(h, residue_index, num_segments=r)
    sums = s_all[:, :256]
    cnts = s_all[:, 256:257]
    res_x = sums / jnp.maximum(cnts, 1.0)
    h2 = jax.nn.relu(res_x @ lin1_W + lin1_b)
    z = h2 @ lin2_W + lin2_b
    x_s = z[src_idx]
    x_t = z[tgt_idx]
    y = 2.0 * y_raw.astype(jnp.float32) - 1.0
    eps = 1e-8
    ns = jnp.maximum(jnp.linalg.norm(x_s, axis=1), eps)
    nt = jnp.maximum(jnp.linalg.norm(x_t, axis=1), eps)
    cos = jnp.sum(x_s * x_t, axis=1) / (ns * nt)
    loss = jnp.mean(jnp.where(y > 0, 1.0 - cos, jnp.maximum(cos, 0.0)))
    return (loss, cos, y)


# speedup vs baseline: 1.1274x; 1.1274x over previous
import functools

import jax
import jax.numpy as jnp
from jax import lax
from jax.experimental import pallas as pl
from jax.experimental.pallas import tpu as pltpu
from jax.experimental.pallas import tpu_sc as plsc

_NC, _NS = 2, 16
_NW = _NC * _NS
_KCH = 400
_BK = 16000


def _sc_gather(xp, pd, src, dst):
    e = src.shape[0]
    e_per_w = e // _NW
    nch = e_per_w // _KCH
    mesh = plsc.VectorSubcoreMesh(core_axis_name="c", subcore_axis_name="s")

    @functools.partial(
        pl.kernel,
        out_type=jax.ShapeDtypeStruct((e, 64), jnp.float32),
        mesh=mesh,
        compiler_params=pltpu.CompilerParams(use_tc_tiling_on_sc=False),
        scratch_types=[
            pltpu.VMEM((_KCH,), jnp.int32),
            pltpu.VMEM((_KCH,), jnp.int32),
            pltpu.VMEM((_KCH, 48), jnp.float32),
            pltpu.VMEM((_KCH, 16), jnp.float32),
            pltpu.SemaphoreType.DMA,
            pltpu.SemaphoreType.DMA,
        ],
    )
    def k(xp_hbm, pd_hbm, src_hbm, dst_hbm, g_out,
          sidx, didx, srows, drows, sem1, sem2):
        wid = lax.axis_index("s") * _NC + lax.axis_index("c")
        base = wid * e_per_w

        def body(j, carry):
            off = base + j * _KCH
            pltpu.sync_copy(src_hbm.at[pl.ds(off, _KCH)], sidx)
            pltpu.sync_copy(dst_hbm.at[pl.ds(off, _KCH)], didx)
            cp1 = pltpu.async_copy(xp_hbm.at[sidx], srows, sem1)
            cp2 = pltpu.async_copy(pd_hbm.at[didx], drows, sem2)
            cp1.wait()
            cp2.wait()
            pltpu.sync_copy(srows, g_out.at[pl.ds(off, _KCH), pl.ds(0, 48)])
            pltpu.sync_copy(drows, g_out.at[pl.ds(off, _KCH), pl.ds(48, 16)])
            return carry

        lax.fori_loop(0, nch, body, 0)

    return k(xp, pd, src, dst)


def _edge_mlp_body(g_ref, w1_ref, b1_ref, w2_ref, b2_ref, o_ref):
    g = jnp.maximum(
        jnp.dot(g_ref[...], w1_ref[...], preferred_element_type=jnp.float32)
        + b1_ref[...], 0.0)
    o = jnp.dot(g, w2_ref[...], preferred_element_type=jnp.float32)
    o_ref[...] = jnp.maximum(o + b2_ref[...], 0.0)


def _edge_mlp(g, Wcat, ba, Wb, bb):
    e = g.shape[0]
    return pl.pallas_call(
        _edge_mlp_body,
        grid=(e // _BK,),
        in_specs=[
            pl.BlockSpec((_BK, 64), lambda i: (i, 0)),
            pl.BlockSpec((64, 64), lambda i: (0, 0)),
            pl.BlockSpec((1, 64), lambda i: (0, 0)),
            pl.BlockSpec((64, 128), lambda i: (0, 0)),
            pl.BlockSpec((1, 128), lambda i: (0, 0)),
        ],
        out_specs=pl.BlockSpec((_BK, 128), lambda i: (i, 0)),
        out_shape=jax.ShapeDtypeStruct((e, 128), jnp.float32),
    )(g, Wcat, ba.reshape(1, 64), Wb, bb.reshape(1, 128))


def _point_conv(xp, pd, edge_index, Wa, ba, Wb, bb, n):
    src = edge_index[0]
    dst = edge_index[1]
    e = src.shape[0]
    step = _NW * _KCH
    e_pad = ((e + step - 1) // step) * step
    dst_seg = dst
    if e_pad != e:
        src = jnp.pad(src, (0, e_pad - e))
        dst = jnp.pad(dst, (0, e_pad - e))
        dst_seg = jnp.pad(dst_seg, (0, e_pad - e), constant_values=n)
    Wcat = jnp.zeros((64, 64), jnp.float32)
    Wcat = Wcat.at[0:32].set(Wa[:32])
    Wcat = Wcat.at[32:35].set(Wa[32:])
    Wcat = Wcat.at[48:51].set(-Wa[32:])
    g = _sc_gather(xp, pd, src, dst)
    msg = _edge_mlp(g, Wcat, ba, Wb, bb)
    out = jax.ops.segment_max(msg, dst_seg, num_segments=n + 1)[:n]
    return jnp.where(jnp.isfinite(out), out, 0.0)


def kernel(x, pos, edge_index1, edge_index2, residue_index, src_idx, tgt_idx,
           y_raw, W1a, b1a, W1b, b1b, W2a, b2a, W2b, b2b, lin1_W, lin1_b,
           lin2_W, lin2_b):
    n = x.shape[0]
    xp = jnp.concatenate([x, pos, jnp.zeros((n, 13), jnp.float32)], axis=1)
    pd = jnp.concatenate([pos, jnp.zeros((n, 13), jnp.float32)], axis=1)
    x1 = _point_conv(xp, pd, edge_index1, W1a, b1a, W1b, b1b, n)
    x2 = _point_conv(xp, pd, edge_index2, W2a, b2a, W2b, b2b, n)
    h = jnp.concatenate([x1, x2, jnp.ones((n, 1), jnp.float32)], axis=1)
    r = 10000
    s_all = jax.ops.segment_sum(h, residue_index, num_segments=r)
    sums = s_all[:, :256]
    cnts = s_all[:, 256:257]
    res_x = sums / jnp.maximum(cnts, 1.0)
    h2 = jax.nn.relu(res_x @ lin1_W + lin1_b)
    z = h2 @ lin2_W + lin2_b
    x_s = z[src_idx]
    x_t = z[tgt_idx]
    y = 2.0 * y_raw.astype(jnp.float32) - 1.0
    eps = 1e-8
    ns = jnp.maximum(jnp.linalg.norm(x_s, axis=1), eps)
    nt = jnp.maximum(jnp.linalg.norm(x_t, axis=1), eps)
    cos = jnp.sum(x_s * x_t, axis=1) / (ns * nt)
    loss = jnp.mean(jnp.where(y > 0, 1.0 - cos, jnp.maximum(cos, 0.0)))
    return (loss, cos, y)


# speedup vs baseline: 1.1366x; 1.0082x over previous
import functools

import jax
import jax.numpy as jnp
from jax import lax
from jax.experimental import pallas as pl
from jax.experimental.pallas import tpu as pltpu
from jax.experimental.pallas import tpu_sc as plsc

_NC, _NS = 2, 16
_NW = _NC * _NS
_KCH = 800
_BK = 16000


def _sc_gather(xp, pd, src, dst):
    e = src.shape[0]
    e_per_w = e // _NW
    nch = e_per_w // _KCH
    mesh = plsc.VectorSubcoreMesh(core_axis_name="c", subcore_axis_name="s")

    @functools.partial(
        pl.kernel,
        out_type=jax.ShapeDtypeStruct((e, 64), jnp.float32),
        mesh=mesh,
        compiler_params=pltpu.CompilerParams(use_tc_tiling_on_sc=False),
        scratch_types=[
            pltpu.VMEM((_KCH,), jnp.int32),
            pltpu.VMEM((_KCH,), jnp.int32),
            pltpu.VMEM((_KCH, 48), jnp.float32),
            pltpu.VMEM((_KCH, 16), jnp.float32),
            pltpu.SemaphoreType.DMA,
            pltpu.SemaphoreType.DMA,
        ],
    )
    def k(xp_hbm, pd_hbm, src_hbm, dst_hbm, g_out,
          sidx, didx, srows, drows, sem1, sem2):
        wid = lax.axis_index("s") * _NC + lax.axis_index("c")
        base = wid * e_per_w

        def body(j, carry):
            off = base + j * _KCH
            pltpu.sync_copy(src_hbm.at[pl.ds(off, _KCH)], sidx)
            pltpu.sync_copy(dst_hbm.at[pl.ds(off, _KCH)], didx)
            cp1 = pltpu.async_copy(xp_hbm.at[sidx], srows, sem1)
            cp2 = pltpu.async_copy(pd_hbm.at[didx], drows, sem2)
            cp1.wait()
            cp2.wait()
            pltpu.sync_copy(srows, g_out.at[pl.ds(off, _KCH), pl.ds(0, 48)])
            pltpu.sync_copy(drows, g_out.at[pl.ds(off, _KCH), pl.ds(48, 16)])
            return carry

        lax.fori_loop(0, nch, body, 0)

    return k(xp, pd, src, dst)


def _edge_mlp_body(g_ref, w1_ref, b1_ref, w2_ref, b2_ref, o_ref):
    g = jnp.maximum(
        jnp.dot(g_ref[...], w1_ref[...], preferred_element_type=jnp.float32)
        + b1_ref[...], 0.0)
    o = jnp.dot(g, w2_ref[...], preferred_element_type=jnp.float32)
    o_ref[...] = jnp.maximum(o + b2_ref[...], 0.0)


def _edge_mlp(g, Wcat, ba, Wb, bb):
    e = g.shape[0]
    return pl.pallas_call(
        _edge_mlp_body,
        grid=(e // _BK,),
        in_specs=[
            pl.BlockSpec((_BK, 64), lambda i: (i, 0)),
            pl.BlockSpec((64, 64), lambda i: (0, 0)),
            pl.BlockSpec((1, 64), lambda i: (0, 0)),
            pl.BlockSpec((64, 128), lambda i: (0, 0)),
            pl.BlockSpec((1, 128), lambda i: (0, 0)),
        ],
        out_specs=pl.BlockSpec((_BK, 128), lambda i: (i, 0)),
        out_shape=jax.ShapeDtypeStruct((e, 128), jnp.float32),
    )(g, Wcat, ba.reshape(1, 64), Wb, bb.reshape(1, 128))


def _point_conv(xp, pd, edge_index, Wa, ba, Wb, bb, n):
    src = edge_index[0]
    dst = edge_index[1]
    e = src.shape[0]
    step = _NW * _KCH
    e_pad = ((e + step - 1) // step) * step
    dst_seg = dst
    if e_pad != e:
        src = jnp.pad(src, (0, e_pad - e))
        dst = jnp.pad(dst, (0, e_pad - e))
        dst_seg = jnp.pad(dst_seg, (0, e_pad - e), constant_values=n)
    Wcat = jnp.zeros((64, 64), jnp.float32)
    Wcat = Wcat.at[0:32].set(Wa[:32])
    Wcat = Wcat.at[32:35].set(Wa[32:])
    Wcat = Wcat.at[48:51].set(-Wa[32:])
    g = _sc_gather(xp, pd, src, dst)
    msg = _edge_mlp(g, Wcat, ba, Wb, bb)
    out = jax.ops.segment_max(msg, dst_seg, num_segments=n + 1)[:n]
    return jnp.where(jnp.isfinite(out), out, 0.0)


def kernel(x, pos, edge_index1, edge_index2, residue_index, src_idx, tgt_idx,
           y_raw, W1a, b1a, W1b, b1b, W2a, b2a, W2b, b2b, lin1_W, lin1_b,
           lin2_W, lin2_b):
    n = x.shape[0]
    xp = jnp.concatenate([x, pos, jnp.zeros((n, 13), jnp.float32)], axis=1)
    pd = jnp.concatenate([pos, jnp.zeros((n, 13), jnp.float32)], axis=1)
    x1 = _point_conv(xp, pd, edge_index1, W1a, b1a, W1b, b1b, n)
    x2 = _point_conv(xp, pd, edge_index2, W2a, b2a, W2b, b2b, n)
    h = jnp.concatenate([x1, x2, jnp.ones((n, 1), jnp.float32)], axis=1)
    r = 10000
    s_all = jax.ops.segment_sum(h, residue_index, num_segments=r)
    sums = s_all[:, :256]
    cnts = s_all[:, 256:257]
    res_x = sums / jnp.maximum(cnts, 1.0)
    h2 = jax.nn.relu(res_x @ lin1_W + lin1_b)
    z = h2 @ lin2_W + lin2_b
    x_s = z[src_idx]
    x_t = z[tgt_idx]
    y = 2.0 * y_raw.astype(jnp.float32) - 1.0
    eps = 1e-8
    ns = jnp.maximum(jnp.linalg.norm(x_s, axis=1), eps)
    nt = jnp.maximum(jnp.linalg.norm(x_t, axis=1), eps)
    cos = jnp.sum(x_s * x_t, axis=1) / (ns * nt)
    loss = jnp.mean(jnp.where(y > 0, 1.0 - cos, jnp.maximum(cos, 0.0)))
    return (loss, cos, y)
